# baseline (speedup 1.0000x reference)
import functools

import jax
import jax.numpy as jnp
from jax import lax
from jax.experimental import pallas as pl
from jax.experimental.pallas import tpu as pltpu
from jax.experimental.pallas import tpu_sc as plsc

N = 2048
D = 768
F = 1536
E = 64
TM = 128
TG = 64
G_MAX = N // TG + E - 1
G_PAD = 128
P_MAX = G_MAX * TG
NW = 32
CHUNK = N // NW


NB = N // TM


def _router_body(x_ref, wr_ref, br_ref,
                 dest_ref, win_ref, se_ref, st_ref, vd_ref,
                 oh_ref, cs_ref, tot_ref):
    b = pl.program_id(0)

    @pl.when(b == 0)
    def _():
        tot_ref[...] = jnp.zeros((1, E), jnp.float32)

    x = x_ref[...]
    logits = lax.dot_general(x, wr_ref[...], (((1,), (1,)), ((), ())),
                             preferred_element_type=jnp.float32) + br_ref[...]
    lmax = jnp.max(logits, axis=1, keepdims=True)
    ee = jnp.exp(logits - lmax)
    probs = ee / jnp.sum(ee, axis=1, keepdims=True)
    pmax = jnp.max(probs, axis=1, keepdims=True)
    eids = lax.broadcasted_iota(jnp.int32, (TM, E), 1)
    idx = jnp.min(jnp.where(probs >= pmax, eids, E), axis=1, keepdims=True)
    onehot = (eids == idx).astype(jnp.float32)
    oh_ref[pl.ds(b * TM, TM), :] = onehot
    lane = lax.broadcasted_iota(jnp.int32, (TM, 128), 1)
    win_ref[pl.ds(b * TM, TM), :] = jnp.where(
        lane == 0, jnp.broadcast_to(pmax, (TM, 128)), 0.0)

    tri = (lax.broadcasted_iota(jnp.int32, (TM, TM), 1)
           <= lax.broadcasted_iota(jnp.int32, (TM, TM), 0)).astype(jnp.float32)
    c = lax.dot_general(tri, onehot, (((1,), (0,)), ((), ())),
                        preferred_element_type=jnp.float32) + tot_ref[...]
    cs_ref[pl.ds(b * TM, TM), :] = c
    tot_ref[...] = c[TM - 1:TM, :]

    @pl.when(b == NB - 1)
    def _():
        onehot_all = oh_ref[...]
        rank = jnp.sum(onehot_all * cs_ref[...], axis=1, keepdims=True) - 1.0

        counts = tot_ref[...].astype(jnp.int32)
        tiles_e = lax.shift_right_logical(counts + (TG - 1), 6)
        triE = (lax.broadcasted_iota(jnp.int32, (E, E), 0)
                <= lax.broadcasted_iota(jnp.int32, (E, E), 1)).astype(jnp.float32)
        bounds = lax.dot_general(tiles_e.astype(jnp.float32), triE,
                                 (((1,), (0,)), ((), ())),
                                 preferred_element_type=jnp.float32)
        n_tiles = bounds[0, E - 1].astype(jnp.int32)
        pad_off = (bounds - tiles_e.astype(jnp.float32)) * float(TG)
        dest = jnp.sum(onehot_all * pad_off, axis=1, keepdims=True) + rank
        dest_ref[...] = dest.astype(jnp.int32)

        g_col = lax.broadcasted_iota(jnp.int32, (G_PAD, 1), 0)
        st = jnp.minimum(g_col, n_tiles - 1)
        st_ref[...] = st
        vd_ref[...] = (g_col < n_tiles).astype(jnp.int32)
        se_ref[...] = jnp.sum((bounds <= st.astype(jnp.float32))
                              .astype(jnp.int32), axis=1, keepdims=True)


_router = pl.pallas_call(
    _router_body,
    grid=(NB,),
    in_specs=[
        pl.BlockSpec((TM, D), lambda b: (b, 0)),
        pl.BlockSpec((E, D), lambda b: (0, 0)),
        pl.BlockSpec((1, E), lambda b: (0, 0)),
    ],
    out_specs=(
        pl.BlockSpec((N, 1), lambda b: (0, 0)),
        pl.BlockSpec((N, 128), lambda b: (0, 0)),
        pl.BlockSpec((G_PAD, 1), lambda b: (0, 0)),
        pl.BlockSpec((G_PAD, 1), lambda b: (0, 0)),
        pl.BlockSpec((G_PAD, 1), lambda b: (0, 0)),
    ),
    out_shape=(
        jax.ShapeDtypeStruct((N, 1), jnp.int32),
        jax.ShapeDtypeStruct((N, 128), jnp.float32),
        jax.ShapeDtypeStruct((G_PAD, 1), jnp.int32),
        jax.ShapeDtypeStruct((G_PAD, 1), jnp.int32),
        jax.ShapeDtypeStruct((G_PAD, 1), jnp.int32),
    ),
    scratch_shapes=[pltpu.VMEM((N, E), jnp.float32),
                    pltpu.VMEM((N, E), jnp.float32),
                    pltpu.VMEM((1, E), jnp.float32)],
    compiler_params=pltpu.CompilerParams(
        dimension_semantics=("arbitrary",)),
)


def _dispatch_body(x_hbm, dest_hbm, win_hbm, xpad_hbm, wpad_hbm,
                   idx_v, rows_v, wrow_v, sem):
    wid = lax.axis_index("s") * 2 + lax.axis_index("c")
    base = wid * CHUNK
    pltpu.sync_copy(dest_hbm.at[pl.ds(base, CHUNK)], idx_v)
    pltpu.sync_copy(x_hbm.at[pl.ds(base, CHUNK)], rows_v)
    pltpu.sync_copy(win_hbm.at[pl.ds(base, CHUNK)], wrow_v)
    c1 = pltpu.async_copy(rows_v, xpad_hbm.at[idx_v], sem)
    c2 = pltpu.async_copy(wrow_v, wpad_hbm.at[idx_v], sem)
    c1.wait()
    c2.wait()


_dispatch = functools.partial(
    pl.kernel,
    mesh=plsc.VectorSubcoreMesh(core_axis_name="c", subcore_axis_name="s"),
    out_type=(jax.ShapeDtypeStruct((P_MAX, D), jnp.float32),
              jax.ShapeDtypeStruct((P_MAX, 128), jnp.float32)),
    scratch_types=[pltpu.VMEM((CHUNK,), jnp.int32),
                   pltpu.VMEM((CHUNK, D), jnp.float32),
                   pltpu.VMEM((CHUNK, 128), jnp.float32),
                   pltpu.SemaphoreType.DMA],
)(_dispatch_body)


def _gmm_body(se_ref, st_ref, vd_ref, x_ref, wp_ref, w1_ref, b1_ref, w2_ref,
              b2_ref, o_ref):
    @pl.when(vd_ref[pl.program_id(0)] == 1)
    def _():
        xg = x_ref[...]
        h = jnp.maximum(
            lax.dot_general(xg, w1_ref[0], (((1,), (1,)), ((), ())),
                            preferred_element_type=jnp.float32)
            + b1_ref[0], 0.0)
        eo = lax.dot_general(h, w2_ref[0], (((1,), (1,)), ((), ())),
                             preferred_element_type=jnp.float32) + b2_ref[0]
        o_ref[...] = eo * wp_ref[:, 0:1]


_gmm = pl.pallas_call(
    _gmm_body,
    grid_spec=pltpu.PrefetchScalarGridSpec(
        num_scalar_prefetch=3,
        grid=(G_MAX,),
        in_specs=[
            pl.BlockSpec((TG, D), lambda g, se, st, vd: (st[g], 0)),
            pl.BlockSpec((TG, 128), lambda g, se, st, vd: (st[g], 0)),
            pl.BlockSpec((1, F, D), lambda g, se, st, vd: (se[g], 0, 0)),
            pl.BlockSpec((1, 1, F), lambda g, se, st, vd: (se[g], 0, 0)),
            pl.BlockSpec((1, D, F), lambda g, se, st, vd: (se[g], 0, 0)),
            pl.BlockSpec((1, 1, D), lambda g, se, st, vd: (se[g], 0, 0)),
        ],
        out_specs=pl.BlockSpec((TG, D), lambda g, se, st, vd: (st[g], 0)),
    ),
    out_shape=jax.ShapeDtypeStruct((P_MAX, D), jnp.float32),
    compiler_params=pltpu.CompilerParams(
        dimension_semantics=("arbitrary",)),
)


def _combine_body(opad_hbm, dest_hbm, y_hbm, idx_v, rows_v, sem):
    wid = lax.axis_index("s") * 2 + lax.axis_index("c")
    base = wid * CHUNK
    pltpu.sync_copy(dest_hbm.at[pl.ds(base, CHUNK)], idx_v)
    pltpu.async_copy(opad_hbm.at[idx_v], rows_v, sem).wait()
    pltpu.sync_copy(rows_v, y_hbm.at[pl.ds(base, CHUNK)])


_combine = functools.partial(
    pl.kernel,
    mesh=plsc.VectorSubcoreMesh(core_axis_name="c", subcore_axis_name="s"),
    out_type=jax.ShapeDtypeStruct((N, D), jnp.float32),
    scratch_types=[pltpu.VMEM((CHUNK,), jnp.int32),
                   pltpu.VMEM((CHUNK, D), jnp.float32),
                   pltpu.SemaphoreType.DMA],
)(_combine_body)


def kernel(x, Wr, br, W1, b1, W2, b2):
    dest2, win, se2, st2, vd2 = _router(x, Wr, br.reshape(1, E))
    dest = dest2.reshape(N)
    xpad, wpad = _dispatch(x, dest, win)
    opad = _gmm(se2.reshape(G_PAD), st2.reshape(G_PAD), vd2.reshape(G_PAD),
                xpad, wpad, W1, b1.reshape(E, 1, F), W2, b2.reshape(E, 1, D))
    return _combine(opad, dest)


# speedup vs baseline: 1.0411x; 1.0411x over previous
import functools

import jax
import jax.numpy as jnp
from jax import lax
from jax.experimental import pallas as pl
from jax.experimental.pallas import tpu as pltpu
from jax.experimental.pallas import tpu_sc as plsc

N = 2048
D = 768
F = 1536
E = 64
TM = 128
TG = 64
G_MAX = N // TG + E - 1
G_PAD = 128
P_MAX = G_MAX * TG
NW = 32
CHUNK = N // NW


def _router_body(x_ref, wr_ref, br_ref,
                 dest_ref, win_ref, se_ref, st_ref, vd_ref,
                 oh_ref, cs_ref):
    x = x_ref[...]
    logits = lax.dot_general(x, wr_ref[...], (((1,), (1,)), ((), ())),
                             preferred_element_type=jnp.float32) + br_ref[...]
    lmax = jnp.max(logits, axis=1, keepdims=True)
    ee = jnp.exp(logits - lmax)
    probs = ee / jnp.sum(ee, axis=1, keepdims=True)
    pmax = jnp.max(probs, axis=1, keepdims=True)
    eids = lax.broadcasted_iota(jnp.int32, (N, E), 1)
    idx = jnp.min(jnp.where(probs >= pmax, eids, E), axis=1, keepdims=True)
    onehot = (eids == idx).astype(jnp.float32)
    oh_ref[...] = onehot

    tri = (lax.broadcasted_iota(jnp.int32, (TM, TM), 1)
           <= lax.broadcasted_iota(jnp.int32, (TM, TM), 0)).astype(jnp.float32)

    def blk(b, tot):
        oh = oh_ref[pl.ds(b * TM, TM), :]
        c = lax.dot_general(tri, oh, (((1,), (0,)), ((), ())),
                            preferred_element_type=jnp.float32) + tot
        cs_ref[pl.ds(b * TM, TM), :] = c
        return c[TM - 1:TM, :]

    tot = lax.fori_loop(0, N // TM, blk, jnp.zeros((1, E), jnp.float32))
    rank = jnp.sum(onehot * cs_ref[...], axis=1, keepdims=True) - 1.0

    counts = tot.astype(jnp.int32)
    tiles_e = lax.shift_right_logical(counts + (TG - 1), 6)
    triE = (lax.broadcasted_iota(jnp.int32, (E, E), 0)
            <= lax.broadcasted_iota(jnp.int32, (E, E), 1)).astype(jnp.float32)
    bounds = lax.dot_general(tiles_e.astype(jnp.float32), triE,
                             (((1,), (0,)), ((), ())),
                             preferred_element_type=jnp.float32)
    n_tiles = bounds[0, E - 1].astype(jnp.int32)
    pad_off = (bounds - tiles_e.astype(jnp.float32)) * float(TG)
    dest = jnp.sum(onehot * pad_off, axis=1, keepdims=True) + rank
    dest_ref[...] = dest.astype(jnp.int32)
    win_ref[...] = jnp.broadcast_to(pmax, (N, 128))

    g_col = lax.broadcasted_iota(jnp.int32, (G_PAD, 1), 0)
    st = jnp.minimum(g_col, n_tiles - 1)
    st_ref[...] = st
    vd_ref[...] = (g_col < n_tiles).astype(jnp.int32)
    se_ref[...] = jnp.sum((bounds <= st.astype(jnp.float32))
                          .astype(jnp.int32), axis=1, keepdims=True)


_router = pl.pallas_call(
    _router_body,
    out_shape=(
        jax.ShapeDtypeStruct((N, 1), jnp.int32),
        jax.ShapeDtypeStruct((N, 128), jnp.float32),
        jax.ShapeDtypeStruct((G_PAD, 1), jnp.int32),
        jax.ShapeDtypeStruct((G_PAD, 1), jnp.int32),
        jax.ShapeDtypeStruct((G_PAD, 1), jnp.int32),
    ),
    scratch_shapes=[pltpu.VMEM((N, E), jnp.float32),
                    pltpu.VMEM((N, E), jnp.float32)],
)


NCH = 4
SCH = CHUNK // NCH


def _dispatch_body(x_hbm, dest_hbm, win_hbm, xpad_hbm, wpad_hbm,
                   idx_v, rows_v, wrow_v, dsem, wsem, ssem, *xsems):
    wid = lax.axis_index("s") * 2 + lax.axis_index("c")
    base = wid * CHUNK
    ld = pltpu.async_copy(dest_hbm.at[pl.ds(wid * NCH, NCH)], idx_v, dsem)
    lw = pltpu.async_copy(win_hbm.at[pl.ds(base, CHUNK)], wrow_v, wsem)
    lx = [pltpu.async_copy(x_hbm.at[pl.ds(base + k * SCH, SCH)],
                           rows_v.at[pl.ds(k * SCH, SCH)], xsems[k])
          for k in range(NCH)]
    ld.wait()
    lw.wait()
    sc = []
    for k in range(NCH):
        sc.append(pltpu.async_copy(wrow_v.at[pl.ds(k * SCH, SCH)],
                                   wpad_hbm.at[idx_v.at[k]], ssem))
    for k in range(NCH):
        lx[k].wait()
        sc.append(pltpu.async_copy(rows_v.at[pl.ds(k * SCH, SCH)],
                                   xpad_hbm.at[idx_v.at[k]], ssem))
    for c in sc:
        c.wait()


_dispatch = functools.partial(
    pl.kernel,
    mesh=plsc.VectorSubcoreMesh(core_axis_name="c", subcore_axis_name="s"),
    out_type=(jax.ShapeDtypeStruct((P_MAX, D), jnp.float32),
              jax.ShapeDtypeStruct((P_MAX, 128), jnp.float32)),
    scratch_types=[pltpu.VMEM((NCH, SCH), jnp.int32),
                   pltpu.VMEM((CHUNK, D), jnp.float32),
                   pltpu.VMEM((CHUNK, 128), jnp.float32),
                   pltpu.SemaphoreType.DMA,
                   pltpu.SemaphoreType.DMA,
                   pltpu.SemaphoreType.DMA,
                   pltpu.SemaphoreType.DMA,
                   pltpu.SemaphoreType.DMA,
                   pltpu.SemaphoreType.DMA,
                   pltpu.SemaphoreType.DMA],
)(_dispatch_body)


def _gmm_body(se_ref, st_ref, vd_ref, x_ref, wp_ref, w1_ref, b1_ref, w2_ref,
              b2_ref, o_ref):
    @pl.when(vd_ref[pl.program_id(0)] == 1)
    def _():
        xg = x_ref[...]
        h = jnp.maximum(
            lax.dot_general(xg, w1_ref[0], (((1,), (1,)), ((), ())),
                            preferred_element_type=jnp.float32)
            + b1_ref[0], 0.0)
        eo = lax.dot_general(h, w2_ref[0], (((1,), (1,)), ((), ())),
                             preferred_element_type=jnp.float32) + b2_ref[0]
        o_ref[...] = eo * wp_ref[:, 0:1]


_gmm = pl.pallas_call(
    _gmm_body,
    grid_spec=pltpu.PrefetchScalarGridSpec(
        num_scalar_prefetch=3,
        grid=(G_MAX,),
        in_specs=[
            pl.BlockSpec((TG, D), lambda g, se, st, vd: (st[g], 0)),
            pl.BlockSpec((TG, 128), lambda g, se, st, vd: (st[g], 0)),
            pl.BlockSpec((1, F, D), lambda g, se, st, vd: (se[g], 0, 0)),
            pl.BlockSpec((1, 1, F), lambda g, se, st, vd: (se[g], 0, 0)),
            pl.BlockSpec((1, D, F), lambda g, se, st, vd: (se[g], 0, 0)),
            pl.BlockSpec((1, 1, D), lambda g, se, st, vd: (se[g], 0, 0)),
        ],
        out_specs=pl.BlockSpec((TG, D), lambda g, se, st, vd: (st[g], 0)),
    ),
    out_shape=jax.ShapeDtypeStruct((P_MAX, D), jnp.float32),
    compiler_params=pltpu.CompilerParams(
        dimension_semantics=("arbitrary",)),
)


def _combine_body(opad_hbm, dest_hbm, y_hbm, idx_v, rows_v, ssem, *gsems):
    wid = lax.axis_index("s") * 2 + lax.axis_index("c")
    base = wid * CHUNK
    pltpu.sync_copy(dest_hbm.at[pl.ds(wid * NCH, NCH)], idx_v)
    gs = [pltpu.async_copy(opad_hbm.at[idx_v.at[k]],
                           rows_v.at[pl.ds(k * SCH, SCH)], gsems[k])
          for k in range(NCH)]
    st = []
    for k in range(NCH):
        gs[k].wait()
        st.append(pltpu.async_copy(rows_v.at[pl.ds(k * SCH, SCH)],
                                   y_hbm.at[pl.ds(base + k * SCH, SCH)], ssem))
    for c in st:
        c.wait()


_combine = functools.partial(
    pl.kernel,
    mesh=plsc.VectorSubcoreMesh(core_axis_name="c", subcore_axis_name="s"),
    out_type=jax.ShapeDtypeStruct((N, D), jnp.float32),
    scratch_types=[pltpu.VMEM((NCH, SCH), jnp.int32),
                   pltpu.VMEM((CHUNK, D), jnp.float32),
                   pltpu.SemaphoreType.DMA,
                   pltpu.SemaphoreType.DMA,
                   pltpu.SemaphoreType.DMA,
                   pltpu.SemaphoreType.DMA,
                   pltpu.SemaphoreType.DMA],
)(_combine_body)


def kernel(x, Wr, br, W1, b1, W2, b2):
    dest2, win, se2, st2, vd2 = _router(x, Wr, br.reshape(1, E))
    destc = dest2.reshape(N // SCH, SCH)
    xpad, wpad = _dispatch(x, destc, win)
    opad = _gmm(se2.reshape(G_PAD), st2.reshape(G_PAD), vd2.reshape(G_PAD),
                xpad, wpad, W1, b1.reshape(E, 1, F), W2, b2.reshape(E, 1, D))
    return _combine(opad, destc)


# speedup vs baseline: 1.0434x; 1.0022x over previous
import functools

import jax
import jax.numpy as jnp
from jax import lax
from jax.experimental import pallas as pl
from jax.experimental.pallas import tpu as pltpu
from jax.experimental.pallas import tpu_sc as plsc

N = 2048
D = 768
F = 1536
E = 64
TM = 128
TG = 64
G_MAX = N // TG + E - 1
G_PAD = 128
P_MAX = G_MAX * TG
NW = 32
CHUNK = N // NW


def _router_body(x_ref, wr_ref, br_ref,
                 dest_ref, win_ref, se_ref, st_ref, vd_ref,
                 oh_ref, cs_ref):
    x = x_ref[...]
    logits = lax.dot_general(x, wr_ref[...], (((1,), (1,)), ((), ())),
                             preferred_element_type=jnp.float32) + br_ref[...]
    lmax = jnp.max(logits, axis=1, keepdims=True)
    ee = jnp.exp(logits - lmax)
    probs = ee / jnp.sum(ee, axis=1, keepdims=True)
    pmax = jnp.max(probs, axis=1, keepdims=True)
    eids = lax.broadcasted_iota(jnp.int32, (N, E), 1)
    idx = jnp.min(jnp.where(probs >= pmax, eids, E), axis=1, keepdims=True)
    onehot = (eids == idx).astype(jnp.float32)
    oh_ref[...] = onehot

    tri = (lax.broadcasted_iota(jnp.int32, (TM, TM), 1)
           <= lax.broadcasted_iota(jnp.int32, (TM, TM), 0)).astype(jnp.float32)

    def blk(b, tot):
        oh = oh_ref[pl.ds(b * TM, TM), :]
        c = lax.dot_general(tri, oh, (((1,), (0,)), ((), ())),
                            preferred_element_type=jnp.float32) + tot
        cs_ref[pl.ds(b * TM, TM), :] = c
        return c[TM - 1:TM, :]

    tot = lax.fori_loop(0, N // TM, blk, jnp.zeros((1, E), jnp.float32))
    rank = jnp.sum(onehot * cs_ref[...], axis=1, keepdims=True) - 1.0

    counts = tot.astype(jnp.int32)
    tiles_e = lax.shift_right_logical(counts + (TG - 1), 6)
    triE = (lax.broadcasted_iota(jnp.int32, (E, E), 0)
            <= lax.broadcasted_iota(jnp.int32, (E, E), 1)).astype(jnp.float32)
    bounds = lax.dot_general(tiles_e.astype(jnp.float32), triE,
                             (((1,), (0,)), ((), ())),
                             preferred_element_type=jnp.float32)
    n_tiles = bounds[0, E - 1].astype(jnp.int32)
    pad_off = (bounds - tiles_e.astype(jnp.float32)) * float(TG)
    dest = jnp.sum(onehot * pad_off, axis=1, keepdims=True) + rank
    dest_ref[...] = dest.astype(jnp.int32)
    win_ref[...] = jnp.broadcast_to(pmax, (N, 128))

    g_col = lax.broadcasted_iota(jnp.int32, (G_PAD, 1), 0)
    st = jnp.minimum(g_col, n_tiles - 1)
    st_ref[...] = st
    vd_ref[...] = (g_col < n_tiles).astype(jnp.int32)
    se_ref[...] = jnp.sum((bounds <= st.astype(jnp.float32))
                          .astype(jnp.int32), axis=1, keepdims=True)


_router = pl.pallas_call(
    _router_body,
    out_shape=(
        jax.ShapeDtypeStruct((N, 1), jnp.int32),
        jax.ShapeDtypeStruct((N, 128), jnp.float32),
        jax.ShapeDtypeStruct((G_PAD, 1), jnp.int32),
        jax.ShapeDtypeStruct((G_PAD, 1), jnp.int32),
        jax.ShapeDtypeStruct((G_PAD, 1), jnp.int32),
    ),
    scratch_shapes=[pltpu.VMEM((N, E), jnp.float32),
                    pltpu.VMEM((N, E), jnp.float32)],
)


NCH = 2
SCH = CHUNK // NCH


def _dispatch_body(x_hbm, dest_hbm, win_hbm, xpad_hbm, wpad_hbm,
                   idx_v, rows_v, wrow_v, dsem, wsem, ssem, *xsems):
    wid = lax.axis_index("s") * 2 + lax.axis_index("c")
    base = wid * CHUNK
    ld = pltpu.async_copy(dest_hbm.at[pl.ds(wid * NCH, NCH)], idx_v, dsem)
    lw = pltpu.async_copy(win_hbm.at[pl.ds(base, CHUNK)], wrow_v, wsem)
    lx = [pltpu.async_copy(x_hbm.at[pl.ds(base + k * SCH, SCH)],
                           rows_v.at[pl.ds(k * SCH, SCH)], xsems[k])
          for k in range(NCH)]
    ld.wait()
    sc = []
    for k in range(NCH):
        lx[k].wait()
        sc.append(pltpu.async_copy(rows_v.at[pl.ds(k * SCH, SCH)],
                                   xpad_hbm.at[idx_v.at[k]], ssem))
    lw.wait()
    for k in range(NCH):
        sc.append(pltpu.async_copy(wrow_v.at[pl.ds(k * SCH, SCH)],
                                   wpad_hbm.at[idx_v.at[k]], ssem))
    for c in sc:
        c.wait()


_dispatch = functools.partial(
    pl.kernel,
    mesh=plsc.VectorSubcoreMesh(core_axis_name="c", subcore_axis_name="s"),
    out_type=(jax.ShapeDtypeStruct((P_MAX, D), jnp.float32),
              jax.ShapeDtypeStruct((P_MAX, 128), jnp.float32)),
    scratch_types=[pltpu.VMEM((NCH, SCH), jnp.int32),
                   pltpu.VMEM((CHUNK, D), jnp.float32),
                   pltpu.VMEM((CHUNK, 128), jnp.float32),
                   pltpu.SemaphoreType.DMA,
                   pltpu.SemaphoreType.DMA,
                   pltpu.SemaphoreType.DMA,
                   pltpu.SemaphoreType.DMA,
                   pltpu.SemaphoreType.DMA],
)(_dispatch_body)


def _gmm_body(se_ref, st_ref, vd_ref, x_ref, wp_ref, w1_ref, b1_ref, w2_ref,
              b2_ref, o_ref):
    @pl.when(vd_ref[pl.program_id(0)] == 1)
    def _():
        xg = x_ref[...]
        h = jnp.maximum(
            lax.dot_general(xg, w1_ref[0], (((1,), (1,)), ((), ())),
                            preferred_element_type=jnp.float32)
            + b1_ref[0], 0.0)
        eo = lax.dot_general(h, w2_ref[0], (((1,), (1,)), ((), ())),
                             preferred_element_type=jnp.float32) + b2_ref[0]
        o_ref[...] = eo * wp_ref[:, 0:1]


_gmm = pl.pallas_call(
    _gmm_body,
    grid_spec=pltpu.PrefetchScalarGridSpec(
        num_scalar_prefetch=3,
        grid=(G_MAX,),
        in_specs=[
            pl.BlockSpec((TG, D), lambda g, se, st, vd: (st[g], 0)),
            pl.BlockSpec((TG, 128), lambda g, se, st, vd: (st[g], 0)),
            pl.BlockSpec((1, F, D), lambda g, se, st, vd: (se[g], 0, 0)),
            pl.BlockSpec((1, 1, F), lambda g, se, st, vd: (se[g], 0, 0)),
            pl.BlockSpec((1, D, F), lambda g, se, st, vd: (se[g], 0, 0)),
            pl.BlockSpec((1, 1, D), lambda g, se, st, vd: (se[g], 0, 0)),
        ],
        out_specs=pl.BlockSpec((TG, D), lambda g, se, st, vd: (st[g], 0)),
    ),
    out_shape=jax.ShapeDtypeStruct((P_MAX, D), jnp.float32),
    compiler_params=pltpu.CompilerParams(
        dimension_semantics=("arbitrary",)),
)


def _combine_body(opad_hbm, dest_hbm, y_hbm, idx_v, rows_v, ssem, *gsems):
    wid = lax.axis_index("s") * 2 + lax.axis_index("c")
    base = wid * CHUNK
    pltpu.sync_copy(dest_hbm.at[pl.ds(wid * NCH, NCH)], idx_v)
    gs = [pltpu.async_copy(opad_hbm.at[idx_v.at[k]],
                           rows_v.at[pl.ds(k * SCH, SCH)], gsems[k])
          for k in range(NCH)]
    st = []
    for k in range(NCH):
        gs[k].wait()
        st.append(pltpu.async_copy(rows_v.at[pl.ds(k * SCH, SCH)],
                                   y_hbm.at[pl.ds(base + k * SCH, SCH)], ssem))
    for c in st:
        c.wait()


_combine = functools.partial(
    pl.kernel,
    mesh=plsc.VectorSubcoreMesh(core_axis_name="c", subcore_axis_name="s"),
    out_type=jax.ShapeDtypeStruct((N, D), jnp.float32),
    scratch_types=[pltpu.VMEM((NCH, SCH), jnp.int32),
                   pltpu.VMEM((CHUNK, D), jnp.float32),
                   pltpu.SemaphoreType.DMA,
                   pltpu.SemaphoreType.DMA,
                   pltpu.SemaphoreType.DMA],
)(_combine_body)


def kernel(x, Wr, br, W1, b1, W2, b2):
    dest2, win, se2, st2, vd2 = _router(x, Wr, br.reshape(1, E))
    destc = dest2.reshape(N // SCH, SCH)
    xpad, wpad = _dispatch(x, destc, win)
    opad = _gmm(se2.reshape(G_PAD), st2.reshape(G_PAD), vd2.reshape(G_PAD),
                xpad, wpad, W1, b1.reshape(E, 1, F), W2, b2.reshape(E, 1, D))
    return _combine(opad, destc)
